# Initial kernel scaffold; baseline (speedup 1.0000x reference)
import jax
import jax.numpy as jnp
from jax.experimental import pallas as pl


def kernel(x, wq, wk, wv, w_proj):
    raise NotImplementedError("write your pallas kernel here")


# speedup vs baseline: 1.7693x; 1.7693x over previous
import jax
import jax.numpy as jnp
from jax.experimental import pallas as pl
from jax.experimental.pallas import tpu as pltpu

S, D, H, DK, DV = 4096, 1024, 16, 64, 64
BM = 512
BQ = 256
SCALE = 1.0 / (DK ** 0.5)


def _qkv_kernel(x_ref, w_ref, q_ref, k_ref, v_ref):
    r = jnp.dot(x_ref[...], w_ref[...], preferred_element_type=jnp.float32)
    q_ref[...] = r[:, :H * DK]
    k_ref[...] = r[:, H * DK:2 * H * DK]
    v_ref[...] = r[:, 2 * H * DK:]


def _attn_kernel(q_ref, k_ref, v_ref, wp_ref, o_ref):
    outs = []
    for h in range(H):
        q = q_ref[:, h * DK:(h + 1) * DK]
        k = k_ref[:, h * DK:(h + 1) * DK]
        s = jax.lax.dot_general(q, k, (((1,), (1,)), ((), ())),
                                preferred_element_type=jnp.float32) * SCALE
        m = jnp.max(s, axis=-1, keepdims=True)
        p = jnp.exp(s - m)
        p = p / jnp.sum(p, axis=-1, keepdims=True)
        outs.append(jnp.dot(p, v_ref[:, h * DV:(h + 1) * DV],
                            preferred_element_type=jnp.float32))
    concat = jnp.concatenate(outs, axis=-1)
    o_ref[...] = jnp.dot(concat, wp_ref[...],
                         preferred_element_type=jnp.float32)


def kernel(x, wq, wk, wv, w_proj):
    wq2 = wq.transpose(1, 0, 2).reshape(D, H * DK)
    wk2 = wk.transpose(1, 0, 2).reshape(D, H * DK)
    wv2 = wv.transpose(1, 0, 2).reshape(D, H * DV)
    w_all = jnp.concatenate([wq2, wk2, wv2], axis=1)

    q_all, k_all, v_all = pl.pallas_call(
        _qkv_kernel,
        grid=(S // BM,),
        in_specs=[
            pl.BlockSpec((BM, D), lambda i: (i, 0)),
            pl.BlockSpec((D, 3 * H * DK), lambda i: (0, 0)),
        ],
        out_specs=[
            pl.BlockSpec((BM, H * DK), lambda i: (i, 0)),
            pl.BlockSpec((BM, H * DK), lambda i: (i, 0)),
            pl.BlockSpec((BM, H * DV), lambda i: (i, 0)),
        ],
        out_shape=[
            jax.ShapeDtypeStruct((S, H * DK), jnp.float32),
            jax.ShapeDtypeStruct((S, H * DK), jnp.float32),
            jax.ShapeDtypeStruct((S, H * DV), jnp.float32),
        ],
        compiler_params=pltpu.CompilerParams(
            dimension_semantics=("parallel",)),
    )(x, w_all)

    return pl.pallas_call(
        _attn_kernel,
        grid=(S // BQ,),
        in_specs=[
            pl.BlockSpec((BQ, H * DK), lambda i: (i, 0)),
            pl.BlockSpec((S, H * DK), lambda i: (0, 0)),
            pl.BlockSpec((S, H * DV), lambda i: (0, 0)),
            pl.BlockSpec((H * DV, D), lambda i: (0, 0)),
        ],
        out_specs=pl.BlockSpec((BQ, D), lambda i: (i, 0)),
        out_shape=jax.ShapeDtypeStruct((S, D), jnp.float32),
        compiler_params=pltpu.CompilerParams(
            dimension_semantics=("parallel",)),
    )(q_all, k_all, v_all, w_proj)


# speedup vs baseline: 2.5003x; 1.4132x over previous
import jax
import jax.numpy as jnp
from jax.experimental import pallas as pl
from jax.experimental.pallas import tpu as pltpu

S, D, H, DK, DV = 4096, 1024, 16, 64, 64
BM = 512
BQ = 256
SCALE = 1.0 / (DK ** 0.5)


def _qkv_kernel(x_ref, w_ref, q_ref, k_ref, v_ref):
    r = jnp.dot(x_ref[...], w_ref[...], preferred_element_type=jnp.float32)
    q_ref[...] = r[:, :H * DK]
    k_ref[...] = r[:, H * DK:2 * H * DK]
    v_ref[...] = r[:, 2 * H * DK:]


def _attn_kernel(q_ref, k_ref, v_ref, wp_ref, o_ref):
    c2 = SCALE * 1.4426950408889634
    outs = []
    for h in range(H):
        q = q_ref[:, h * DK:(h + 1) * DK]
        k = k_ref[:, h * DK:(h + 1) * DK]
        s = jax.lax.dot_general(q, k, (((1,), (1,)), ((), ())),
                                preferred_element_type=jnp.float32)
        m = jnp.max(s, axis=-1, keepdims=True)
        p = jnp.exp2((s - m) * c2)
        pv = jnp.dot(p, v_ref[:, h * DV:(h + 1) * DV],
                     preferred_element_type=jnp.float32)
        outs.append(pv / jnp.sum(p, axis=-1, keepdims=True))
    concat = jnp.concatenate(outs, axis=-1)
    o_ref[...] = jnp.dot(concat, wp_ref[...],
                         preferred_element_type=jnp.float32)


def kernel(x, wq, wk, wv, w_proj):
    wq2 = wq.transpose(1, 0, 2).reshape(D, H * DK)
    wk2 = wk.transpose(1, 0, 2).reshape(D, H * DK)
    wv2 = wv.transpose(1, 0, 2).reshape(D, H * DV)
    w_all = jnp.concatenate([wq2, wk2, wv2], axis=1)

    q_all, k_all, v_all = pl.pallas_call(
        _qkv_kernel,
        grid=(S // BM,),
        in_specs=[
            pl.BlockSpec((BM, D), lambda i: (i, 0)),
            pl.BlockSpec((D, 3 * H * DK), lambda i: (0, 0)),
        ],
        out_specs=[
            pl.BlockSpec((BM, H * DK), lambda i: (i, 0)),
            pl.BlockSpec((BM, H * DK), lambda i: (i, 0)),
            pl.BlockSpec((BM, H * DV), lambda i: (i, 0)),
        ],
        out_shape=[
            jax.ShapeDtypeStruct((S, H * DK), jnp.float32),
            jax.ShapeDtypeStruct((S, H * DK), jnp.float32),
            jax.ShapeDtypeStruct((S, H * DV), jnp.float32),
        ],
        compiler_params=pltpu.CompilerParams(
            dimension_semantics=("parallel",)),
    )(x, w_all)

    return pl.pallas_call(
        _attn_kernel,
        grid=(S // BQ,),
        in_specs=[
            pl.BlockSpec((BQ, H * DK), lambda i: (i, 0)),
            pl.BlockSpec((S, H * DK), lambda i: (0, 0)),
            pl.BlockSpec((S, H * DV), lambda i: (0, 0)),
            pl.BlockSpec((H * DV, D), lambda i: (0, 0)),
        ],
        out_specs=pl.BlockSpec((BQ, D), lambda i: (i, 0)),
        out_shape=jax.ShapeDtypeStruct((S, D), jnp.float32),
        compiler_params=pltpu.CompilerParams(
            dimension_semantics=("parallel",)),
    )(q_all, k_all, v_all, w_proj)
